# baseline (device time: 120175 ns/iter reference)
import jax
import jax.numpy as jnp
from jax import lax
from jax.experimental import pallas as pl
from jax.experimental.pallas import tpu as pltpu

N_DEV = 16

_RING = [0, 4, 8, 12, 13, 9, 5, 1, 2, 6, 10, 14, 15, 11, 7, 3]
_POS = [0] * N_DEV
for _r, _p in enumerate(_RING):
    _POS[_p] = _r
_RIGHT = [0] * N_DEV
_LEFT = [0] * N_DEV
for _r, _p in enumerate(_RING):
    _RIGHT[_p] = _RING[(_r + 1) % N_DEV]
    _LEFT[_p] = _RING[(_r - 1) % N_DEV]


def _scalar_map(idx, table):
    import jax.numpy as jnp
    out = jnp.int32(table[0])
    for i in range(1, len(table)):
        out = jnp.where(idx == i, jnp.int32(table[i]), out)
    return out


def kernel(x, W1, W2):
    m, k = x.shape
    _, h = W1.shape
    _, n = W2.shape
    chunk = m // N_DEV
    half = chunk // 2

    def body(x_ref, w1_ref, w2_ref, out_ref,
             ptop, pbot, ctop, cbot,
             sp_s, sp_r, sm_s, sm_r, ap_s, ap_r, am_s, am_r):
        me = lax.axis_index("i")
        my = _scalar_map(me, _POS)
        right = _scalar_map(me, _RIGHT)
        left = _scalar_map(me, _LEFT)

        barrier_sem = pltpu.get_barrier_semaphore()
        for nbr in [left, right]:
            pl.semaphore_signal(
                barrier_sem, inc=1,
                device_id=(nbr,), device_id_type=pl.DeviceIdType.MESH,
            )
        pl.semaphore_wait(barrier_sem, 2)

        def compute_top(c):
            xs = x_ref[pl.ds(c * chunk, half), :]
            hm = jnp.maximum(
                jnp.dot(xs, w1_ref[...], preferred_element_type=jnp.float32), 0.0)
            ptop[c] = jnp.dot(hm, w2_ref[...], preferred_element_type=jnp.float32)

        def compute_bot(c):
            xs = x_ref[pl.ds(c * chunk + half, half), :]
            hm = jnp.maximum(
                jnp.dot(xs, w1_ref[...], preferred_element_type=jnp.float32), 0.0)
            pbot[c] = jnp.dot(hm, w2_ref[...], preferred_element_type=jnp.float32)

        compute_top(my)
        compute_bot(my)
        compute_top((my - 1) % N_DEV)
        compute_bot((my + 1) % N_DEV)

        for s in range(N_DEV - 1):
            st = (my - s) % N_DEV
            rt = (my - s - 1) % N_DEV
            sb = (my + s) % N_DEV
            rb = (my + s + 1) % N_DEV
            rp = pltpu.make_async_remote_copy(
                src_ref=ptop.at[st], dst_ref=ctop.at[s],
                send_sem=sp_s.at[s], recv_sem=sp_r.at[s],
                device_id=(right,), device_id_type=pl.DeviceIdType.MESH,
            )
            rm = pltpu.make_async_remote_copy(
                src_ref=pbot.at[sb], dst_ref=cbot.at[s],
                send_sem=sm_s.at[s], recv_sem=sm_r.at[s],
                device_id=(left,), device_id_type=pl.DeviceIdType.MESH,
            )
            rp.start()
            rm.start()
            if s < N_DEV - 2:
                compute_top((my - s - 2) % N_DEV)
                compute_bot((my + s + 2) % N_DEV)
            rp.wait_recv()
            rm.wait_recv()
            ptop[rt] = ptop[rt] + ctop[s]
            pbot[rb] = pbot[rb] + cbot[s]

        for s in range(N_DEV - 1):
            st = (my - s) % N_DEV
            sb = (my + s) % N_DEV
            pltpu.make_async_remote_copy(
                src_ref=ptop.at[st], dst_ref=ctop.at[s],
                send_sem=sp_s.at[s], recv_sem=sp_r.at[s],
                device_id=(right,), device_id_type=pl.DeviceIdType.MESH,
            ).wait_send()
            pltpu.make_async_remote_copy(
                src_ref=pbot.at[sb], dst_ref=cbot.at[s],
                send_sem=sm_s.at[s], recv_sem=sm_r.at[s],
                device_id=(left,), device_id_type=pl.DeviceIdType.MESH,
            ).wait_send()

        ownt = (my + 1) % N_DEV
        ownb = (my - 1) % N_DEV
        out_ref[pl.ds(ownt * chunk, half), :] = ptop[ownt]
        out_ref[pl.ds(ownb * chunk + half, half), :] = pbot[ownb]

        for t in range(N_DEV - 1):
            ct_ = (my + 1 - t) % N_DEV
            cb_ = (my - 1 + t) % N_DEV
            rp = pltpu.make_async_remote_copy(
                src_ref=out_ref.at[pl.ds(ct_ * chunk, half), :],
                dst_ref=out_ref.at[pl.ds(ct_ * chunk, half), :],
                send_sem=ap_s.at[t], recv_sem=ap_r.at[t],
                device_id=(right,), device_id_type=pl.DeviceIdType.MESH,
            )
            rm = pltpu.make_async_remote_copy(
                src_ref=out_ref.at[pl.ds(cb_ * chunk + half, half), :],
                dst_ref=out_ref.at[pl.ds(cb_ * chunk + half, half), :],
                send_sem=am_s.at[t], recv_sem=am_r.at[t],
                device_id=(left,), device_id_type=pl.DeviceIdType.MESH,
            )
            rp.start()
            rm.start()
            rp.wait_recv()
            rm.wait_recv()

        for t in range(N_DEV - 1):
            ct_ = (my + 1 - t) % N_DEV
            cb_ = (my - 1 + t) % N_DEV
            pltpu.make_async_remote_copy(
                src_ref=out_ref.at[pl.ds(ct_ * chunk, half), :],
                dst_ref=out_ref.at[pl.ds(ct_ * chunk, half), :],
                send_sem=ap_s.at[t], recv_sem=ap_r.at[t],
                device_id=(right,), device_id_type=pl.DeviceIdType.MESH,
            ).wait_send()
            pltpu.make_async_remote_copy(
                src_ref=out_ref.at[pl.ds(cb_ * chunk + half, half), :],
                dst_ref=out_ref.at[pl.ds(cb_ * chunk + half, half), :],
                send_sem=am_s.at[t], recv_sem=am_r.at[t],
                device_id=(left,), device_id_type=pl.DeviceIdType.MESH,
            ).wait_send()

    nsteps = N_DEV - 1
    return pl.pallas_call(
        body,
        out_shape=jax.ShapeDtypeStruct((m, n), jnp.float32),
        in_specs=[
            pl.BlockSpec(memory_space=pltpu.VMEM),
            pl.BlockSpec(memory_space=pltpu.VMEM),
            pl.BlockSpec(memory_space=pltpu.VMEM),
        ],
        out_specs=pl.BlockSpec(memory_space=pltpu.VMEM),
        scratch_shapes=[
            pltpu.VMEM((N_DEV, half, n), jnp.float32),
            pltpu.VMEM((N_DEV, half, n), jnp.float32),
            pltpu.VMEM((nsteps, half, n), jnp.float32),
            pltpu.VMEM((nsteps, half, n), jnp.float32),
            pltpu.SemaphoreType.DMA((nsteps,)),
            pltpu.SemaphoreType.DMA((nsteps,)),
            pltpu.SemaphoreType.DMA((nsteps,)),
            pltpu.SemaphoreType.DMA((nsteps,)),
            pltpu.SemaphoreType.DMA((nsteps,)),
            pltpu.SemaphoreType.DMA((nsteps,)),
            pltpu.SemaphoreType.DMA((nsteps,)),
            pltpu.SemaphoreType.DMA((nsteps,)),
        ],
        compiler_params=pltpu.CompilerParams(collective_id=0),
    )(x, W1, W2)


# device time: 89849 ns/iter; 1.3375x vs baseline; 1.3375x over previous
import jax
import jax.numpy as jnp
from jax import lax
from jax.experimental import pallas as pl
from jax.experimental.pallas import tpu as pltpu

N_DEV = 16
NQ = 4


def kernel(x, W1, W2):
    m, k = x.shape
    _, h = W1.shape
    _, n = W2.shape
    chunk = m // NQ
    half = chunk // 2

    def body(x_ref, w1_ref, w2_ref, out_ref,
             ptop, pbot, ctop, cbot, zb1, zb2,
             sp_s, sp_r, sm_s, sm_r,
             za_s, za_r, zb_s, zb_r,
             ga_s, ga_r, gb_s, gb_r,
             ap_s, ap_r, am_s, am_r):
        me = lax.axis_index("i")
        z = me // NQ
        q = me % NQ
        right = z * NQ + (q + 1) % NQ
        left = z * NQ + (q - 1) % NQ
        w1p = me ^ 4
        w2p = me ^ 8

        barrier_sem = pltpu.get_barrier_semaphore()
        for nbr in [left, right, w1p, w2p]:
            pl.semaphore_signal(
                barrier_sem, inc=1,
                device_id=(nbr,), device_id_type=pl.DeviceIdType.MESH,
            )
        pl.semaphore_wait(barrier_sem, 4)

        def compute_top(c):
            xs = x_ref[pl.ds(c * chunk, half), :]
            hm = jnp.maximum(
                jnp.dot(xs, w1_ref[...], preferred_element_type=jnp.float32), 0.0)
            ptop[c] = jnp.dot(hm, w2_ref[...], preferred_element_type=jnp.float32)

        def compute_bot(c):
            xs = x_ref[pl.ds(c * chunk + half, half), :]
            hm = jnp.maximum(
                jnp.dot(xs, w1_ref[...], preferred_element_type=jnp.float32), 0.0)
            pbot[c] = jnp.dot(hm, w2_ref[...], preferred_element_type=jnp.float32)

        compute_top(q)
        compute_bot(q)

        for s in range(NQ - 1):
            st = (q - s) % NQ
            rt = (q - s - 1) % NQ
            sb = (q + s) % NQ
            rb = (q + s + 1) % NQ
            rp = pltpu.make_async_remote_copy(
                src_ref=ptop.at[st], dst_ref=ctop.at[s],
                send_sem=sp_s.at[s], recv_sem=sp_r.at[s],
                device_id=(right,), device_id_type=pl.DeviceIdType.MESH,
            )
            rm = pltpu.make_async_remote_copy(
                src_ref=pbot.at[sb], dst_ref=cbot.at[s],
                send_sem=sm_s.at[s], recv_sem=sm_r.at[s],
                device_id=(left,), device_id_type=pl.DeviceIdType.MESH,
            )
            rp.start()
            rm.start()
            if s == 0:
                compute_top((q - 1) % NQ)
                compute_bot((q + 1) % NQ)
            if s < NQ - 2:
                compute_top((q - s - 2) % NQ)
                compute_bot((q + s + 2) % NQ)
            rp.wait_recv()
            rm.wait_recv()
            ptop[rt] = ptop[rt] + ctop[s]
            pbot[rb] = pbot[rb] + cbot[s]
            rp.wait_send()
            rm.wait_send()

        ownt = (q + 1) % NQ
        ownb = (q - 1) % NQ

        koff = jnp.where(z % 2 == 0, 0, half // 2)
        soff = (half // 2) - koff
        e1t = pltpu.make_async_remote_copy(
            src_ref=ptop.at[ownt, pl.ds(soff, half // 2), :],
            dst_ref=zb1.at[0],
            send_sem=za_s.at[0], recv_sem=za_r.at[0],
            device_id=(w1p,), device_id_type=pl.DeviceIdType.MESH,
        )
        e1b = pltpu.make_async_remote_copy(
            src_ref=pbot.at[ownb, pl.ds(soff, half // 2), :],
            dst_ref=zb1.at[1],
            send_sem=za_s.at[1], recv_sem=za_r.at[1],
            device_id=(w1p,), device_id_type=pl.DeviceIdType.MESH,
        )
        e1t.start()
        e1b.start()
        e1t.wait_recv()
        e1b.wait_recv()
        ptop[ownt, pl.ds(koff, half // 2), :] = (
            ptop[ownt, pl.ds(koff, half // 2), :] + zb1[0])
        pbot[ownb, pl.ds(koff, half // 2), :] = (
            pbot[ownb, pl.ds(koff, half // 2), :] + zb1[1])
        e1t.wait_send()
        e1b.wait_send()

        k2 = jnp.where((z // 2) % 2 == 0, 0, half // 4)
        s2 = (half // 4) - k2
        e2t = pltpu.make_async_remote_copy(
            src_ref=ptop.at[ownt, pl.ds(koff + s2, half // 4), :],
            dst_ref=zb2.at[0],
            send_sem=zb_s.at[0], recv_sem=zb_r.at[0],
            device_id=(w2p,), device_id_type=pl.DeviceIdType.MESH,
        )
        e2b = pltpu.make_async_remote_copy(
            src_ref=pbot.at[ownb, pl.ds(koff + s2, half // 4), :],
            dst_ref=zb2.at[1],
            send_sem=zb_s.at[1], recv_sem=zb_r.at[1],
            device_id=(w2p,), device_id_type=pl.DeviceIdType.MESH,
        )
        e2t.start()
        e2b.start()
        e2t.wait_recv()
        e2b.wait_recv()
        ptop[ownt, pl.ds(koff + k2, half // 4), :] = (
            ptop[ownt, pl.ds(koff + k2, half // 4), :] + zb2[0])
        pbot[ownb, pl.ds(koff + k2, half // 4), :] = (
            pbot[ownb, pl.ds(koff + k2, half // 4), :] + zb2[1])
        e2t.wait_send()
        e2b.wait_send()

        trow = ownt * chunk + koff + k2
        brow = ownb * chunk + half + koff + k2
        out_ref[pl.ds(trow, half // 4), :] = ptop[ownt, pl.ds(koff + k2, half // 4), :]
        out_ref[pl.ds(brow, half // 4), :] = pbot[ownb, pl.ds(koff + k2, half // 4), :]

        g2t = pltpu.make_async_remote_copy(
            src_ref=out_ref.at[pl.ds(trow, half // 4), :],
            dst_ref=out_ref.at[pl.ds(trow, half // 4), :],
            send_sem=ga_s.at[0], recv_sem=ga_r.at[0],
            device_id=(w2p,), device_id_type=pl.DeviceIdType.MESH,
        )
        g2b = pltpu.make_async_remote_copy(
            src_ref=out_ref.at[pl.ds(brow, half // 4), :],
            dst_ref=out_ref.at[pl.ds(brow, half // 4), :],
            send_sem=ga_s.at[1], recv_sem=ga_r.at[1],
            device_id=(w2p,), device_id_type=pl.DeviceIdType.MESH,
        )
        g2t.start()
        g2b.start()
        g2t.wait_recv()
        g2b.wait_recv()
        g2t.wait_send()
        g2b.wait_send()

        t64 = ownt * chunk + koff
        b64 = ownb * chunk + half + koff
        g1t = pltpu.make_async_remote_copy(
            src_ref=out_ref.at[pl.ds(t64, half // 2), :],
            dst_ref=out_ref.at[pl.ds(t64, half // 2), :],
            send_sem=gb_s.at[0], recv_sem=gb_r.at[0],
            device_id=(w1p,), device_id_type=pl.DeviceIdType.MESH,
        )
        g1b = pltpu.make_async_remote_copy(
            src_ref=out_ref.at[pl.ds(b64, half // 2), :],
            dst_ref=out_ref.at[pl.ds(b64, half // 2), :],
            send_sem=gb_s.at[1], recv_sem=gb_r.at[1],
            device_id=(w1p,), device_id_type=pl.DeviceIdType.MESH,
        )
        g1t.start()
        g1b.start()
        g1t.wait_recv()
        g1b.wait_recv()
        g1t.wait_send()
        g1b.wait_send()

        for t in range(NQ - 1):
            ct_ = (q + 1 - t) % NQ
            cb_ = (q - 1 + t) % NQ
            rp = pltpu.make_async_remote_copy(
                src_ref=out_ref.at[pl.ds(ct_ * chunk, half), :],
                dst_ref=out_ref.at[pl.ds(ct_ * chunk, half), :],
                send_sem=ap_s.at[t], recv_sem=ap_r.at[t],
                device_id=(right,), device_id_type=pl.DeviceIdType.MESH,
            )
            rm = pltpu.make_async_remote_copy(
                src_ref=out_ref.at[pl.ds(cb_ * chunk + half, half), :],
                dst_ref=out_ref.at[pl.ds(cb_ * chunk + half, half), :],
                send_sem=am_s.at[t], recv_sem=am_r.at[t],
                device_id=(left,), device_id_type=pl.DeviceIdType.MESH,
            )
            rp.start()
            rm.start()
            rp.wait_recv()
            rm.wait_recv()
            rp.wait_send()
            rm.wait_send()

    nsteps = NQ - 1
    return pl.pallas_call(
        body,
        out_shape=jax.ShapeDtypeStruct((m, n), jnp.float32),
        in_specs=[
            pl.BlockSpec(memory_space=pltpu.VMEM),
            pl.BlockSpec(memory_space=pltpu.VMEM),
            pl.BlockSpec(memory_space=pltpu.VMEM),
        ],
        out_specs=pl.BlockSpec(memory_space=pltpu.VMEM),
        scratch_shapes=[
            pltpu.VMEM((NQ, half, n), jnp.float32),
            pltpu.VMEM((NQ, half, n), jnp.float32),
            pltpu.VMEM((nsteps, half, n), jnp.float32),
            pltpu.VMEM((nsteps, half, n), jnp.float32),
            pltpu.VMEM((2, half // 2, n), jnp.float32),
            pltpu.VMEM((2, half // 4, n), jnp.float32),
            pltpu.SemaphoreType.DMA((nsteps,)),
            pltpu.SemaphoreType.DMA((nsteps,)),
            pltpu.SemaphoreType.DMA((nsteps,)),
            pltpu.SemaphoreType.DMA((nsteps,)),
            pltpu.SemaphoreType.DMA((2,)),
            pltpu.SemaphoreType.DMA((2,)),
            pltpu.SemaphoreType.DMA((2,)),
            pltpu.SemaphoreType.DMA((2,)),
            pltpu.SemaphoreType.DMA((2,)),
            pltpu.SemaphoreType.DMA((2,)),
            pltpu.SemaphoreType.DMA((2,)),
            pltpu.SemaphoreType.DMA((2,)),
            pltpu.SemaphoreType.DMA((nsteps,)),
            pltpu.SemaphoreType.DMA((nsteps,)),
            pltpu.SemaphoreType.DMA((nsteps,)),
            pltpu.SemaphoreType.DMA((nsteps,)),
        ],
        compiler_params=pltpu.CompilerParams(collective_id=0),
    )(x, W1, W2)
